# baseline (speedup 1.0000x reference)
import functools

import jax
import jax.numpy as jnp
from jax import lax
from jax.experimental import pallas as pl
from jax.experimental.pallas import tpu as pltpu
from jax.experimental.pallas import tpu_sc as plsc

VOCAB = 100000
D_MODEL = 1024
RANK = 16
V1 = 317
V2 = 316
D1 = 32
D2 = 32
KD = RANK * D1

N_TOK = 8192
NC = 2
NS = 16
NW = NC * NS
TOK_PER_W = N_TOK // NW
CH = 128
NCH = TOK_PER_W // CH

DIV_M = 13273
DIV_S = 22

TOK_BLK = 512


def _sc_gather_body(ids_hbm, a_hbm, b_hbm, c1_hbm, c2_hbm,
                    ids_v, idx1_v, idx2_v, rows_v, sem):
    wid = lax.axis_index("s") * NC + lax.axis_index("c")
    base = wid * TOK_PER_W
    pltpu.sync_copy(ids_hbm.at[pl.ds(base, TOK_PER_W)], ids_v)
    for k in range(TOK_PER_W // 16):
        v = ids_v[pl.ds(16 * k, 16)]
        q = lax.shift_right_logical(v * DIV_M, DIV_S)
        r = v - q * V2
        q = jnp.where(r >= V2, q + 1, q)
        r = jnp.where(r >= V2, r - V2, r)
        idx1_v[pl.ds(16 * k, 16)] = jnp.minimum(q, V1 - 1)
        idx2_v[pl.ds(16 * k, 16)] = r
    for c in range(NCH):
        pltpu.async_copy(a_hbm.at[idx1_v.at[pl.ds(c * CH, CH)]], rows_v, sem).wait()
        pltpu.sync_copy(rows_v, c1_hbm.at[pl.ds(base + c * CH, CH)])
        pltpu.async_copy(b_hbm.at[idx2_v.at[pl.ds(c * CH, CH)]], rows_v, sem).wait()
        pltpu.sync_copy(rows_v, c2_hbm.at[pl.ds(base + c * CH, CH)])


_sc_gather = functools.partial(
    pl.kernel,
    _sc_gather_body,
    out_type=[
        jax.ShapeDtypeStruct((N_TOK, KD), jnp.float32),
        jax.ShapeDtypeStruct((N_TOK, KD), jnp.float32),
    ],
    mesh=plsc.VectorSubcoreMesh(
        core_axis_name="c", subcore_axis_name="s",
        num_cores=NC, num_subcores=NS,
    ),
    scratch_types=[
        pltpu.VMEM((TOK_PER_W,), jnp.int32),
        pltpu.VMEM((TOK_PER_W,), jnp.int32),
        pltpu.VMEM((TOK_PER_W,), jnp.int32),
        pltpu.VMEM((CH, KD), jnp.float32),
        pltpu.SemaphoreType.DMA,
    ],
)()


def _tc_body(c1_ref, c2_ref, ph_ref, out_ref, pmb_ref):
    i = pl.program_id(0)

    @pl.when(i == 0)
    def _prep():
        pmb_ref[...] = jnp.broadcast_to(jnp.cos(ph_ref[...]), (KD, TOK_BLK))

    c1 = jnp.clip(c1_ref[...].T * pmb_ref[...], -10.0, 10.0)
    c2 = jnp.clip(c2_ref[...].T, -10.0, 10.0)

    accs = []
    for d in range(D1):
        acc = c1[d : d + 1, :] * c2[0:D2, :]
        for r in range(1, RANK):
            acc = acc + c1[32 * r + d : 32 * r + d + 1, :] * c2[32 * r : 32 * r + D2, :]
        accs.append(acc)
    out_t = jnp.concatenate(accs, axis=0)
    out_ref[...] = out_t.T


@jax.jit
def kernel(input_ids, core1, core2, phase_shift):
    b, l = input_ids.shape
    n_tok = b * l
    n_blk = n_tok // TOK_BLK

    ids = input_ids.reshape(n_tok)
    a = core1.reshape(V1, KD)
    b2 = core2.reshape(V2, KD)
    ph = jnp.repeat(phase_shift, D1).reshape(KD, 1)

    c1g, c2g = _sc_gather(ids, a, b2)

    out = pl.pallas_call(
        _tc_body,
        grid=(n_blk,),
        in_specs=[
            pl.BlockSpec((TOK_BLK, KD), lambda i: (i, 0)),
            pl.BlockSpec((TOK_BLK, KD), lambda i: (i, 0)),
            pl.BlockSpec((KD, 1), lambda i: (0, 0)),
        ],
        out_specs=pl.BlockSpec((TOK_BLK, D_MODEL), lambda i: (i, 0)),
        out_shape=jax.ShapeDtypeStruct((n_tok, D_MODEL), jnp.float32),
        scratch_shapes=[
            pltpu.VMEM((KD, TOK_BLK), jnp.float32),
        ],
        compiler_params=pltpu.CompilerParams(
            dimension_semantics=("arbitrary",),
        ),
    )(c1g, c2g, ph)
    return out.reshape(b, l, D_MODEL)


# speedup vs baseline: 1.0835x; 1.0835x over previous
import functools

import jax
import jax.numpy as jnp
from jax import lax
from jax.experimental import pallas as pl
from jax.experimental.pallas import tpu as pltpu
from jax.experimental.pallas import tpu_sc as plsc

VOCAB = 100000
D_MODEL = 1024
RANK = 16
V1 = 317
V2 = 316
D1 = 32
D2 = 32
KD = RANK * D1

N_TOK = 8192
NQ = 4
Q_TOK = N_TOK // NQ
NC = 2
NS = 16
NW = NC * NS
TOK_PER_W = Q_TOK // NW

DIV_M = 13273
DIV_S = 22

TOK_BLK = 512


def _sc_gather_body(ids_hbm, a_hbm, b_hbm, c1_hbm, c2_hbm,
                    ids_v, idx1_v, idx2_v, rows_v, sem):
    wid = lax.axis_index("s") * NC + lax.axis_index("c")
    base = wid * TOK_PER_W
    pltpu.sync_copy(ids_hbm.at[pl.ds(base, TOK_PER_W)], ids_v)
    for k in range(TOK_PER_W // 16):
        v = ids_v[pl.ds(16 * k, 16)]
        q = lax.shift_right_logical(v * DIV_M, DIV_S)
        r = v - q * V2
        q = jnp.where(r >= V2, q + 1, q)
        r = jnp.where(r >= V2, r - V2, r)
        idx1_v[pl.ds(16 * k, 16)] = jnp.minimum(q, V1 - 1)
        idx2_v[pl.ds(16 * k, 16)] = r
    cp1 = pltpu.async_copy(a_hbm.at[idx1_v], rows_v.at[0], sem)
    cp2 = pltpu.async_copy(b_hbm.at[idx2_v], rows_v.at[1], sem)
    cp1.wait()
    pltpu.sync_copy(rows_v.at[0], c1_hbm.at[pl.ds(base, TOK_PER_W)])
    cp2.wait()
    pltpu.sync_copy(rows_v.at[1], c2_hbm.at[pl.ds(base, TOK_PER_W)])


_sc_gather = functools.partial(
    pl.kernel,
    _sc_gather_body,
    out_type=[
        jax.ShapeDtypeStruct((Q_TOK, KD), jnp.float32),
        jax.ShapeDtypeStruct((Q_TOK, KD), jnp.float32),
    ],
    mesh=plsc.VectorSubcoreMesh(
        core_axis_name="c", subcore_axis_name="s",
        num_cores=NC, num_subcores=NS,
    ),
    scratch_types=[
        pltpu.VMEM((TOK_PER_W,), jnp.int32),
        pltpu.VMEM((TOK_PER_W,), jnp.int32),
        pltpu.VMEM((TOK_PER_W,), jnp.int32),
        pltpu.VMEM((2, TOK_PER_W, KD), jnp.float32),
        pltpu.SemaphoreType.DMA,
    ],
)()


def _tc_body(c1_ref, c2_ref, ph_ref, prev_ref, out_ref, pmb_ref):
    del prev_ref
    i = pl.program_id(0)

    @pl.when(i == 0)
    def _prep():
        pmb_ref[...] = jnp.broadcast_to(jnp.cos(ph_ref[...]), (KD, TOK_BLK))

    c1 = jnp.clip(c1_ref[...].T * pmb_ref[...], -10.0, 10.0)
    c2 = jnp.clip(c2_ref[...].T, -10.0, 10.0)

    accs = []
    for d in range(D1):
        acc = c1[d : d + 1, :] * c2[0:D2, :]
        for r in range(1, RANK):
            acc = acc + c1[32 * r + d : 32 * r + d + 1, :] * c2[32 * r : 32 * r + D2, :]
        accs.append(acc)
    out_t = jnp.concatenate(accs, axis=0)
    out_ref[...] = out_t.T


def _tc_body_first(c1_ref, c2_ref, ph_ref, out_ref, pmb_ref):
    _tc_body(c1_ref, c2_ref, ph_ref, None, out_ref, pmb_ref)


def _tc_combine(c1g, c2g, ph, prev, blk_off):
    n_blk_q = Q_TOK // TOK_BLK
    off = blk_off
    common = dict(
        grid=(n_blk_q,),
        out_specs=pl.BlockSpec((TOK_BLK, D_MODEL), lambda i: (i + off, 0)),
        out_shape=jax.ShapeDtypeStruct((N_TOK, D_MODEL), jnp.float32),
        scratch_shapes=[pltpu.VMEM((KD, TOK_BLK), jnp.float32)],
        compiler_params=pltpu.CompilerParams(
            dimension_semantics=("arbitrary",),
        ),
    )
    data_specs = [
        pl.BlockSpec((TOK_BLK, KD), lambda i: (i, 0)),
        pl.BlockSpec((TOK_BLK, KD), lambda i: (i, 0)),
        pl.BlockSpec((KD, 1), lambda i: (0, 0)),
    ]
    if prev is None:
        return pl.pallas_call(
            _tc_body_first, in_specs=data_specs, **common,
        )(c1g, c2g, ph)
    return pl.pallas_call(
        _tc_body,
        in_specs=data_specs + [pl.BlockSpec(memory_space=pl.ANY)],
        input_output_aliases={3: 0},
        **common,
    )(c1g, c2g, ph, prev)


@jax.jit
def kernel(input_ids, core1, core2, phase_shift):
    b, l = input_ids.shape
    n_tok = b * l

    ids = input_ids.reshape(n_tok)
    a = core1.reshape(V1, KD)
    b2 = core2.reshape(V2, KD)
    ph = jnp.repeat(phase_shift, D1).reshape(KD, 1)

    gathered = [_sc_gather(ids[q * Q_TOK : (q + 1) * Q_TOK], a, b2) for q in range(NQ)]

    buf = None
    for q in range(NQ):
        c1g, c2g = gathered[q]
        buf = _tc_combine(c1g, c2g, ph, buf, q * (Q_TOK // TOK_BLK))
    return buf.reshape(b, l, D_MODEL)


# speedup vs baseline: 1.0857x; 1.0020x over previous
import functools

import jax
import jax.numpy as jnp
from jax import lax
from jax.experimental import pallas as pl
from jax.experimental.pallas import tpu as pltpu
from jax.experimental.pallas import tpu_sc as plsc

VOCAB = 100000
D_MODEL = 1024
RANK = 16
V1 = 317
V2 = 316
D1 = 32
D2 = 32
KD = RANK * D1
VPAD = 320

N_TOK = 8192
NQ = 4
Q_TOK = N_TOK // NQ
NC = 2
NS = 16
NW = NC * NS
TOK_PER_W = Q_TOK // NW

DIV_M = 13273
DIV_S = 22

TOK_BLK = 512
NBQ = Q_TOK // TOK_BLK


def _sc_gather_body(ids_hbm, b_hbm, c2_hbm, ids_v, idx2_v, rows_v, sem):
    wid = lax.axis_index("s") * NC + lax.axis_index("c")
    base = wid * TOK_PER_W
    pltpu.sync_copy(ids_hbm.at[pl.ds(base, TOK_PER_W)], ids_v)
    for k in range(TOK_PER_W // 16):
        v = ids_v[pl.ds(16 * k, 16)]
        q = lax.shift_right_logical(v * DIV_M, DIV_S)
        r = v - q * V2
        r = jnp.where(r >= V2, r - V2, r)
        idx2_v[pl.ds(16 * k, 16)] = r
    pltpu.async_copy(b_hbm.at[idx2_v], rows_v, sem).wait()
    pltpu.sync_copy(rows_v, c2_hbm.at[pl.ds(base, TOK_PER_W)])


_sc_gather = functools.partial(
    pl.kernel,
    _sc_gather_body,
    out_type=[jax.ShapeDtypeStruct((Q_TOK, KD), jnp.float32)],
    mesh=plsc.VectorSubcoreMesh(
        core_axis_name="c", subcore_axis_name="s",
        num_cores=NC, num_subcores=NS,
    ),
    scratch_types=[
        pltpu.VMEM((TOK_PER_W,), jnp.int32),
        pltpu.VMEM((TOK_PER_W,), jnp.int32),
        pltpu.VMEM((TOK_PER_W, KD), jnp.float32),
        pltpu.SemaphoreType.DMA,
    ],
)()


def _tc_body(ids_ref, at_ref, ph_ref, c2_ref, prev_ref, out_ref, ah_ref, al_ref):
    del prev_ref
    i = pl.program_id(0)

    @pl.when(i == 0)
    def _split_tables():
        pm = jnp.cos(ph_ref[...])
        a = jnp.clip(at_ref[...] * pm, -10.0, 10.0)
        ah = a.astype(jnp.bfloat16)
        ah_ref[...] = ah
        al_ref[...] = (a - ah.astype(jnp.float32)).astype(jnp.bfloat16)

    ids = ids_ref[0]
    idx1 = jnp.clip(ids // V2, 0, V1 - 1)

    iota0 = jax.lax.broadcasted_iota(jnp.int32, (VPAD, TOK_BLK), 0)
    oh1 = (iota0 == idx1).astype(jnp.bfloat16)
    c1 = jnp.dot(ah_ref[...], oh1, preferred_element_type=jnp.float32)
    c1 = c1 + jnp.dot(al_ref[...], oh1, preferred_element_type=jnp.float32)

    c2 = jnp.clip(c2_ref[...].T, -10.0, 10.0)

    accs = []
    for d in range(D1):
        acc = c1[d : d + 1, :] * c2[0:D2, :]
        for r in range(1, RANK):
            acc = acc + c1[32 * r + d : 32 * r + d + 1, :] * c2[32 * r : 32 * r + D2, :]
        accs.append(acc)
    out_t = jnp.concatenate(accs, axis=0)
    out_ref[...] = out_t.T


def _tc_body_first(ids_ref, at_ref, ph_ref, c2_ref, out_ref, ah_ref, al_ref):
    _tc_body(ids_ref, at_ref, ph_ref, c2_ref, None, out_ref, ah_ref, al_ref)


def _tc_combine(ids3_q, a_t, ph, c2g, prev, blk_off):
    off = blk_off
    common = dict(
        grid=(NBQ,),
        out_specs=pl.BlockSpec((TOK_BLK, D_MODEL), lambda i: (i + off, 0)),
        out_shape=jax.ShapeDtypeStruct((N_TOK, D_MODEL), jnp.float32),
        scratch_shapes=[
            pltpu.VMEM((KD, VPAD), jnp.bfloat16),
            pltpu.VMEM((KD, VPAD), jnp.bfloat16),
        ],
        compiler_params=pltpu.CompilerParams(
            dimension_semantics=("arbitrary",),
        ),
    )
    data_specs = [
        pl.BlockSpec((1, 1, TOK_BLK), lambda i: (i, 0, 0)),
        pl.BlockSpec((KD, VPAD), lambda i: (0, 0)),
        pl.BlockSpec((KD, 1), lambda i: (0, 0)),
        pl.BlockSpec((TOK_BLK, KD), lambda i: (i, 0)),
    ]
    if prev is None:
        return pl.pallas_call(
            _tc_body_first, in_specs=data_specs, **common,
        )(ids3_q, a_t, ph, c2g)
    return pl.pallas_call(
        _tc_body,
        in_specs=data_specs + [pl.BlockSpec(memory_space=pl.ANY)],
        input_output_aliases={4: 0},
        **common,
    )(ids3_q, a_t, ph, c2g, prev)


@jax.jit
def kernel(input_ids, core1, core2, phase_shift):
    b, l = input_ids.shape
    n_tok = b * l

    ids = input_ids.reshape(n_tok)
    ids3 = input_ids.reshape(n_tok // TOK_BLK, 1, TOK_BLK)
    a_t = jnp.pad(core1.reshape(V1, KD), ((0, VPAD - V1), (0, 0))).T
    b2 = core2.reshape(V2, KD)
    ph = jnp.repeat(phase_shift, D1).reshape(KD, 1)

    gathered = [_sc_gather(ids[q * Q_TOK : (q + 1) * Q_TOK], b2)[0] for q in range(NQ)]

    buf = None
    for q in range(NQ):
        buf = _tc_combine(
            ids3[q * NBQ : (q + 1) * NBQ], a_t, ph, gathered[q], buf, q * NBQ,
        )
    return buf.reshape(b, l, D_MODEL)


# speedup vs baseline: 1.1036x; 1.0166x over previous
import functools

import jax
import jax.numpy as jnp
from jax import lax
from jax.experimental import pallas as pl
from jax.experimental.pallas import tpu as pltpu
from jax.experimental.pallas import tpu_sc as plsc

VOCAB = 100000
D_MODEL = 1024
RANK = 16
V1 = 317
V2 = 316
D1 = 32
D2 = 32
KD = RANK * D1
VPAD = 320

N_TOK = 8192
NQ = 1
Q_TOK = N_TOK // NQ
NC = 2
NS = 16
NW = NC * NS
TOK_PER_W = Q_TOK // NW

DIV_M = 13273
DIV_S = 22

TOK_BLK = 512
NBQ = Q_TOK // TOK_BLK


def _sc_gather_body(ids_hbm, b_hbm, c2_hbm, ids_v, idx2_v, rows_v, sem):
    wid = lax.axis_index("s") * NC + lax.axis_index("c")
    base = wid * TOK_PER_W
    pltpu.sync_copy(ids_hbm.at[pl.ds(base, TOK_PER_W)], ids_v)
    for k in range(TOK_PER_W // 16):
        v = ids_v[pl.ds(16 * k, 16)]
        q = lax.shift_right_logical(v * DIV_M, DIV_S)
        r = v - q * V2
        r = jnp.where(r >= V2, r - V2, r)
        idx2_v[pl.ds(16 * k, 16)] = r
    for c in range(TOK_PER_W // 128):
        pltpu.async_copy(b_hbm.at[idx2_v.at[pl.ds(c * 128, 128)]], rows_v, sem).wait()
        pltpu.sync_copy(rows_v, c2_hbm.at[pl.ds(base + c * 128, 128)])


_sc_gather = functools.partial(
    pl.kernel,
    _sc_gather_body,
    out_type=[jax.ShapeDtypeStruct((Q_TOK, KD), jnp.float32)],
    mesh=plsc.VectorSubcoreMesh(
        core_axis_name="c", subcore_axis_name="s",
        num_cores=NC, num_subcores=NS,
    ),
    scratch_types=[
        pltpu.VMEM((TOK_PER_W,), jnp.int32),
        pltpu.VMEM((TOK_PER_W,), jnp.int32),
        pltpu.VMEM((128, KD), jnp.float32),
        pltpu.SemaphoreType.DMA,
    ],
)()


def _tc_body(ids_ref, at_ref, ph_ref, c2_ref, prev_ref, out_ref, ah_ref, al_ref):
    del prev_ref
    i = pl.program_id(0)

    @pl.when(i == 0)
    def _split_tables():
        pm = jnp.cos(ph_ref[...])
        a = jnp.clip(at_ref[...] * pm, -10.0, 10.0)
        ah = a.astype(jnp.bfloat16)
        ah_ref[...] = ah
        al_ref[...] = (a - ah.astype(jnp.float32)).astype(jnp.bfloat16)

    ids = ids_ref[0]
    idx1 = jnp.clip(ids // V2, 0, V1 - 1)

    iota0 = jax.lax.broadcasted_iota(jnp.int32, (VPAD, TOK_BLK), 0)
    oh1 = (iota0 == idx1).astype(jnp.bfloat16)
    c1 = jnp.dot(ah_ref[...], oh1, preferred_element_type=jnp.float32)
    c1 = c1 + jnp.dot(al_ref[...], oh1, preferred_element_type=jnp.float32)

    c2 = jnp.clip(c2_ref[...].T, -10.0, 10.0)

    accs = []
    for d in range(D1):
        acc = c1[d : d + 1, :] * c2[0:D2, :]
        for r in range(1, RANK):
            acc = acc + c1[32 * r + d : 32 * r + d + 1, :] * c2[32 * r : 32 * r + D2, :]
        accs.append(acc)
    out_t = jnp.concatenate(accs, axis=0)
    out_ref[...] = out_t.T


def _tc_body_first(ids_ref, at_ref, ph_ref, c2_ref, out_ref, ah_ref, al_ref):
    _tc_body(ids_ref, at_ref, ph_ref, c2_ref, None, out_ref, ah_ref, al_ref)


def _tc_combine(ids3_q, a_t, ph, c2g, prev, blk_off):
    off = blk_off
    common = dict(
        grid=(NBQ,),
        out_specs=pl.BlockSpec((TOK_BLK, D_MODEL), lambda i: (i + off, 0)),
        out_shape=jax.ShapeDtypeStruct((N_TOK, D_MODEL), jnp.float32),
        scratch_shapes=[
            pltpu.VMEM((KD, VPAD), jnp.bfloat16),
            pltpu.VMEM((KD, VPAD), jnp.bfloat16),
        ],
        compiler_params=pltpu.CompilerParams(
            dimension_semantics=("arbitrary",),
        ),
    )
    data_specs = [
        pl.BlockSpec((1, 1, TOK_BLK), lambda i: (i, 0, 0)),
        pl.BlockSpec((KD, VPAD), lambda i: (0, 0)),
        pl.BlockSpec((KD, 1), lambda i: (0, 0)),
        pl.BlockSpec((TOK_BLK, KD), lambda i: (i, 0)),
    ]
    if prev is None:
        return pl.pallas_call(
            _tc_body_first, in_specs=data_specs, **common,
        )(ids3_q, a_t, ph, c2g)
    return pl.pallas_call(
        _tc_body,
        in_specs=data_specs + [pl.BlockSpec(memory_space=pl.ANY)],
        input_output_aliases={4: 0},
        **common,
    )(ids3_q, a_t, ph, c2g, prev)


@jax.jit
def kernel(input_ids, core1, core2, phase_shift):
    b, l = input_ids.shape
    n_tok = b * l

    ids = input_ids.reshape(n_tok)
    ids3 = input_ids.reshape(n_tok // TOK_BLK, 1, TOK_BLK)
    a_t = jnp.pad(core1.reshape(V1, KD), ((0, VPAD - V1), (0, 0))).T
    b2 = core2.reshape(V2, KD)
    ph = jnp.repeat(phase_shift, D1).reshape(KD, 1)

    gathered = [_sc_gather(ids[q * Q_TOK : (q + 1) * Q_TOK], b2)[0] for q in range(NQ)]

    buf = None
    for q in range(NQ):
        buf = _tc_combine(
            ids3[q * NBQ : (q + 1) * NBQ], a_t, ph, gathered[q], buf, q * NBQ,
        )
    return buf.reshape(b, l, D_MODEL)


# speedup vs baseline: 1.1251x; 1.0194x over previous
import functools

import jax
import jax.numpy as jnp
from jax import lax
from jax.experimental import pallas as pl
from jax.experimental.pallas import tpu as pltpu
from jax.experimental.pallas import tpu_sc as plsc

VOCAB = 100000
D_MODEL = 1024
RANK = 16
V1 = 317
V2 = 316
D1 = 32
D2 = 32
KD = RANK * D1
VPAD = 320

N_TOK = 8192
NQ = 1
Q_TOK = N_TOK // NQ
NC = 2
NS = 16
NW = NC * NS
TOK_PER_W = Q_TOK // NW

DIV_M = 13273
DIV_S = 22

TOK_BLK = 1024
NBQ = Q_TOK // TOK_BLK


def _sc_gather_body(ids_hbm, b_hbm, c2_hbm, ids_v, idx2_v, rows_v, sem):
    wid = lax.axis_index("s") * NC + lax.axis_index("c")
    base = wid * TOK_PER_W
    pltpu.sync_copy(ids_hbm.at[pl.ds(base, TOK_PER_W)], ids_v)
    for k in range(TOK_PER_W // 16):
        v = ids_v[pl.ds(16 * k, 16)]
        q = lax.shift_right_logical(v * DIV_M, DIV_S)
        r = v - q * V2
        r = jnp.where(r >= V2, r - V2, r)
        idx2_v[pl.ds(16 * k, 16)] = r
    for c in range(TOK_PER_W // 128):
        pltpu.async_copy(b_hbm.at[idx2_v.at[pl.ds(c * 128, 128)]], rows_v, sem).wait()
        pltpu.sync_copy(rows_v, c2_hbm.at[pl.ds(base + c * 128, 128)])


_sc_gather = functools.partial(
    pl.kernel,
    _sc_gather_body,
    out_type=[jax.ShapeDtypeStruct((Q_TOK, KD), jnp.float32)],
    mesh=plsc.VectorSubcoreMesh(
        core_axis_name="c", subcore_axis_name="s",
        num_cores=NC, num_subcores=NS,
    ),
    scratch_types=[
        pltpu.VMEM((TOK_PER_W,), jnp.int32),
        pltpu.VMEM((TOK_PER_W,), jnp.int32),
        pltpu.VMEM((128, KD), jnp.float32),
        pltpu.SemaphoreType.DMA,
    ],
)()


def _tc_body(ids_ref, at_ref, ph_ref, c2_ref, prev_ref, out_ref, ah_ref, al_ref):
    del prev_ref
    i = pl.program_id(0)

    @pl.when(i == 0)
    def _split_tables():
        pm = jnp.cos(ph_ref[...])
        a = jnp.clip(at_ref[...] * pm, -10.0, 10.0)
        ah = a.astype(jnp.bfloat16)
        ah_ref[...] = ah
        al_ref[...] = (a - ah.astype(jnp.float32)).astype(jnp.bfloat16)

    ids = ids_ref[0]
    idx1 = jnp.clip(ids // V2, 0, V1 - 1)

    iota0 = jax.lax.broadcasted_iota(jnp.int32, (VPAD, TOK_BLK), 0)
    oh1 = (iota0 == idx1).astype(jnp.bfloat16)
    c1 = jnp.dot(ah_ref[...], oh1, preferred_element_type=jnp.float32)
    c1 = c1 + jnp.dot(al_ref[...], oh1, preferred_element_type=jnp.float32)

    c2 = jnp.clip(c2_ref[...].T, -10.0, 10.0)

    accs = []
    for d in range(D1):
        acc = c1[d : d + 1, :] * c2[0:D2, :]
        for r in range(1, RANK):
            acc = acc + c1[32 * r + d : 32 * r + d + 1, :] * c2[32 * r : 32 * r + D2, :]
        accs.append(acc)
    out_t = jnp.concatenate(accs, axis=0)
    out_ref[...] = out_t.T


def _tc_body_first(ids_ref, at_ref, ph_ref, c2_ref, out_ref, ah_ref, al_ref):
    _tc_body(ids_ref, at_ref, ph_ref, c2_ref, None, out_ref, ah_ref, al_ref)


def _tc_combine(ids3_q, a_t, ph, c2g, prev, blk_off):
    off = blk_off
    common = dict(
        grid=(NBQ,),
        out_specs=pl.BlockSpec((TOK_BLK, D_MODEL), lambda i: (i + off, 0)),
        out_shape=jax.ShapeDtypeStruct((N_TOK, D_MODEL), jnp.float32),
        scratch_shapes=[
            pltpu.VMEM((KD, VPAD), jnp.bfloat16),
            pltpu.VMEM((KD, VPAD), jnp.bfloat16),
        ],
        compiler_params=pltpu.CompilerParams(
            dimension_semantics=("arbitrary",),
        ),
    )
    data_specs = [
        pl.BlockSpec((1, 1, TOK_BLK), lambda i: (i, 0, 0)),
        pl.BlockSpec((KD, VPAD), lambda i: (0, 0)),
        pl.BlockSpec((KD, 1), lambda i: (0, 0)),
        pl.BlockSpec((TOK_BLK, KD), lambda i: (i, 0)),
    ]
    if prev is None:
        return pl.pallas_call(
            _tc_body_first, in_specs=data_specs, **common,
        )(ids3_q, a_t, ph, c2g)
    return pl.pallas_call(
        _tc_body,
        in_specs=data_specs + [pl.BlockSpec(memory_space=pl.ANY)],
        input_output_aliases={4: 0},
        **common,
    )(ids3_q, a_t, ph, c2g, prev)


@jax.jit
def kernel(input_ids, core1, core2, phase_shift):
    b, l = input_ids.shape
    n_tok = b * l

    ids = input_ids.reshape(n_tok)
    ids3 = input_ids.reshape(n_tok // TOK_BLK, 1, TOK_BLK)
    a_t = jnp.pad(core1.reshape(V1, KD), ((0, VPAD - V1), (0, 0))).T
    b2 = core2.reshape(V2, KD)
    ph = jnp.repeat(phase_shift, D1).reshape(KD, 1)

    gathered = [_sc_gather(ids[q * Q_TOK : (q + 1) * Q_TOK], b2)[0] for q in range(NQ)]

    buf = None
    for q in range(NQ):
        buf = _tc_combine(
            ids3[q * NBQ : (q + 1) * NBQ], a_t, ph, gathered[q], buf, q * NBQ,
        )
    return buf.reshape(b, l, D_MODEL)


# speedup vs baseline: 1.1704x; 1.0403x over previous
import functools

import jax
import jax.numpy as jnp
from jax import lax
from jax.experimental import pallas as pl
from jax.experimental.pallas import tpu as pltpu
from jax.experimental.pallas import tpu_sc as plsc

VOCAB = 100000
D_MODEL = 1024
RANK = 16
V1 = 317
V2 = 316
D1 = 32
D2 = 32
KD = RANK * D1
VPAD = 320

N_TOK = 8192
NQ = 2
Q_TOK = N_TOK // NQ
NC = 2
NS = 16
NW = NC * NS
TOK_PER_W = Q_TOK // NW

DIV_M = 13273
DIV_S = 22

TOK_BLK = 1024
NBQ = Q_TOK // TOK_BLK


def _sc_gather_body(ids_hbm, b_hbm, c2_hbm, ids_v, idx2_v, rows_v, sem):
    wid = lax.axis_index("s") * NC + lax.axis_index("c")
    base = wid * TOK_PER_W
    pltpu.sync_copy(ids_hbm.at[pl.ds(base, TOK_PER_W)], ids_v)
    for k in range(TOK_PER_W // 16):
        v = ids_v[pl.ds(16 * k, 16)]
        q = lax.shift_right_logical(v * DIV_M, DIV_S)
        r = v - q * V2
        r = jnp.where(r >= V2, r - V2, r)
        idx2_v[pl.ds(16 * k, 16)] = r
    for c in range(TOK_PER_W // 128):
        pltpu.async_copy(b_hbm.at[idx2_v.at[pl.ds(c * 128, 128)]], rows_v, sem).wait()
        pltpu.sync_copy(rows_v, c2_hbm.at[pl.ds(base + c * 128, 128)])


_sc_gather = functools.partial(
    pl.kernel,
    _sc_gather_body,
    out_type=[jax.ShapeDtypeStruct((Q_TOK, KD), jnp.float32)],
    mesh=plsc.VectorSubcoreMesh(
        core_axis_name="c", subcore_axis_name="s",
        num_cores=NC, num_subcores=NS,
    ),
    scratch_types=[
        pltpu.VMEM((TOK_PER_W,), jnp.int32),
        pltpu.VMEM((TOK_PER_W,), jnp.int32),
        pltpu.VMEM((128, KD), jnp.float32),
        pltpu.SemaphoreType.DMA,
    ],
)()


def _tc_body(ids_ref, at_ref, ph_ref, c2_ref, prev_ref, out_ref, ah_ref, al_ref):
    del prev_ref
    i = pl.program_id(0)

    @pl.when(i == 0)
    def _split_tables():
        pm = jnp.cos(ph_ref[...])
        a = jnp.clip(at_ref[...] * pm, -10.0, 10.0)
        ah = a.astype(jnp.bfloat16)
        ah_ref[...] = ah
        al_ref[...] = (a - ah.astype(jnp.float32)).astype(jnp.bfloat16)

    ids = ids_ref[0]
    idx1 = jnp.clip(ids // V2, 0, V1 - 1)

    iota0 = jax.lax.broadcasted_iota(jnp.int32, (VPAD, TOK_BLK), 0)
    oh1 = (iota0 == idx1).astype(jnp.bfloat16)
    c1 = jnp.dot(ah_ref[...], oh1, preferred_element_type=jnp.float32)
    c1 = c1 + jnp.dot(al_ref[...], oh1, preferred_element_type=jnp.float32)

    c2 = jnp.clip(c2_ref[...].T, -10.0, 10.0)

    accs = []
    for d in range(D1):
        acc = c1[d : d + 1, :] * c2[0:D2, :]
        for r in range(1, RANK):
            acc = acc + c1[32 * r + d : 32 * r + d + 1, :] * c2[32 * r : 32 * r + D2, :]
        accs.append(acc)
    out_t = jnp.concatenate(accs, axis=0)
    out_ref[...] = out_t.T


def _tc_body_first(ids_ref, at_ref, ph_ref, c2_ref, out_ref, ah_ref, al_ref):
    _tc_body(ids_ref, at_ref, ph_ref, c2_ref, None, out_ref, ah_ref, al_ref)


def _tc_combine(ids3_q, a_t, ph, c2g, prev, blk_off):
    off = blk_off
    common = dict(
        grid=(NBQ,),
        out_specs=pl.BlockSpec((TOK_BLK, D_MODEL), lambda i: (i + off, 0)),
        out_shape=jax.ShapeDtypeStruct((N_TOK, D_MODEL), jnp.float32),
        scratch_shapes=[
            pltpu.VMEM((KD, VPAD), jnp.bfloat16),
            pltpu.VMEM((KD, VPAD), jnp.bfloat16),
        ],
        compiler_params=pltpu.CompilerParams(
            dimension_semantics=("arbitrary",),
        ),
    )
    data_specs = [
        pl.BlockSpec((1, 1, TOK_BLK), lambda i: (i, 0, 0)),
        pl.BlockSpec((KD, VPAD), lambda i: (0, 0)),
        pl.BlockSpec((KD, 1), lambda i: (0, 0)),
        pl.BlockSpec((TOK_BLK, KD), lambda i: (i, 0)),
    ]
    if prev is None:
        return pl.pallas_call(
            _tc_body_first, in_specs=data_specs, **common,
        )(ids3_q, a_t, ph, c2g)
    return pl.pallas_call(
        _tc_body,
        in_specs=data_specs + [pl.BlockSpec(memory_space=pl.ANY)],
        input_output_aliases={4: 0},
        **common,
    )(ids3_q, a_t, ph, c2g, prev)


@jax.jit
def kernel(input_ids, core1, core2, phase_shift):
    b, l = input_ids.shape
    n_tok = b * l

    ids = input_ids.reshape(n_tok)
    ids3 = input_ids.reshape(n_tok // TOK_BLK, 1, TOK_BLK)
    a_t = jnp.pad(core1.reshape(V1, KD), ((0, VPAD - V1), (0, 0))).T
    b2 = core2.reshape(V2, KD)
    ph = jnp.repeat(phase_shift, D1).reshape(KD, 1)

    gathered = [_sc_gather(ids[q * Q_TOK : (q + 1) * Q_TOK], b2)[0] for q in range(NQ)]

    buf = None
    for q in range(NQ):
        buf = _tc_combine(
            ids3[q * NBQ : (q + 1) * NBQ], a_t, ph, gathered[q], buf, q * NBQ,
        )
    return buf.reshape(b, l, D_MODEL)
